# baseline (speedup 1.0000x reference)
import functools

import jax
import jax.numpy as jnp
from jax import lax
from jax.experimental import pallas as pl
from jax.experimental.pallas import tpu as pltpu
from jax.experimental.pallas import tpu_sc as plsc

_N = 7
_MESH = plsc.VectorSubcoreMesh(core_axis_name="c", subcore_axis_name="s")


@functools.partial(
    pl.kernel,
    out_type=jax.ShapeDtypeStruct((16,), jnp.float32),
    mesh=_MESH,
    scratch_types=[
        pltpu.VMEM((16,), jnp.int32),
        pltpu.VMEM((16,), jnp.float32),
        pltpu.SemaphoreType.DMA,
    ],
)
def _sc_gather(x_hbm, vec_hbm, out_hbm, x_v, val_v, sem):
    cid = lax.axis_index("c")
    sid = lax.axis_index("s")

    @pl.when(jnp.logical_and(cid == 0, sid == 0))
    def _():
        pltpu.sync_copy(x_hbm, x_v)
        xv = x_v[...]
        vals = [xv[i] for i in range(_N)]
        acc = jnp.int32(0)
        k = 0
        for i in range(1, _N):
            for j in range(i):
                bit = (vals[i] == vals[j]).astype(jnp.int32)
                acc = acc + bit * jnp.int32(1 << k)
                k += 1
        idx_vec = jnp.full((16,), acc, jnp.int32)
        pltpu.async_copy(vec_hbm.at[idx_vec], val_v, sem).wait()
        pltpu.sync_copy(val_v, out_hbm)


def kernel(x, vec):
    x16 = jnp.zeros((16,), jnp.int32).at[:_N].set(x.astype(jnp.int32))
    out = _sc_gather(x16, vec)
    return out[0]


# speedup vs baseline: 1.0932x; 1.0932x over previous
import functools

import jax
import jax.numpy as jnp
from jax import lax
from jax.experimental import pallas as pl
from jax.experimental.pallas import tpu as pltpu
from jax.experimental.pallas import tpu_sc as plsc

_N = 7
_MESH = plsc.VectorSubcoreMesh(
    core_axis_name="c", subcore_axis_name="s", num_cores=1
)


@functools.partial(
    pl.kernel,
    out_type=jax.ShapeDtypeStruct((1,), jnp.float32),
    mesh=_MESH,
    scratch_types=[
        pltpu.VMEM((16,), jnp.int32),
        pltpu.VMEM((16,), jnp.float32),
        pltpu.SemaphoreType.DMA,
    ],
)
def _sc_gather(x_hbm, vec_hbm, out_hbm, x_v, val_v, sem):
    sid = lax.axis_index("s")

    @pl.when(sid == 0)
    def _():
        pltpu.sync_copy(x_hbm, x_v.at[pl.ds(0, _N)])
        xv = x_v[...]
        vals = [xv[i] for i in range(_N)]
        acc = jnp.int32(0)
        k = 0
        for i in range(1, _N):
            for j in range(i):
                bit = (vals[i] == vals[j]).astype(jnp.int32)
                acc = acc + bit * jnp.int32(1 << k)
                k += 1
        idx_vec = jnp.full((16,), acc, jnp.int32)
        pltpu.async_copy(vec_hbm.at[idx_vec], val_v, sem).wait()
        pltpu.sync_copy(val_v.at[pl.ds(0, 1)], out_hbm)


def kernel(x, vec):
    out = _sc_gather(x.astype(jnp.int32), vec)
    return out.reshape(())
